# baseline (speedup 1.0000x reference)
import functools

import jax
import jax.numpy as jnp
from jax import lax
from jax.experimental import pallas as pl
from jax.experimental.pallas import tpu as pltpu


def _silu(u):
    return u / (1.0 + jnp.exp(-u))


def _sigmoid(u):
    return 1.0 / (1.0 + jnp.exp(-u))


def _dot_nt(a, b):
    return lax.dot_general(a, b, (((1,), (1,)), ((), ())),
                           preferred_element_type=jnp.float32)


def _combined_weights(logits):
    T, E = logits.shape
    lane = lax.broadcasted_iota(jnp.int32, (T, E), 1)
    m1 = jnp.max(logits, axis=1, keepdims=True)
    i1 = jnp.min(jnp.where(logits == m1, lane, E), axis=1, keepdims=True)
    masked = jnp.where(lane == i1, -jnp.inf, logits)
    m2 = jnp.max(masked, axis=1, keepdims=True)
    i2 = jnp.min(jnp.where(masked == m2, lane, E), axis=1, keepdims=True)
    w1 = 1.0 / (1.0 + jnp.exp(m2 - m1))
    w2 = 1.0 - w1
    return jnp.where(lane == i1, w1, 0.0) + jnp.where(lane == i2, w2, 0.0)


def _fused_body(x_ref, gw_ref, wg_ref, wu_ref, wd_ref,
                swg_ref, swu_ref, swd_ref, sg_ref,
                out_ref, cw_ref, *, n_exp, tbs):
    e = pl.program_id(0)
    tb = pl.program_id(1)
    sl = pl.ds(tb * tbs, tbs)

    @pl.when((e == 0) & (tb == 0))
    def _():
        cw_ref[...] = _combined_weights(_dot_nt(x_ref[...], gw_ref[...]))

    xs = x_ref[sl, :]
    xb = xs.astype(jnp.bfloat16)

    @pl.when(e < n_exp)
    def _():
        g = _dot_nt(xb, wg_ref[0].astype(jnp.bfloat16))
        u = _dot_nt(xb, wu_ref[0].astype(jnp.bfloat16))
        h = (g * _silu(u)).astype(jnp.bfloat16)
        y = _dot_nt(h, wd_ref[0].astype(jnp.bfloat16))
        lane = lax.broadcasted_iota(jnp.int32, (tbs, n_exp), 1)
        tokw = jnp.sum(cw_ref[sl, :] * jnp.where(lane == e, 1.0, 0.0),
                       axis=1, keepdims=True)
        contrib = y * tokw

        @pl.when(e == 0)
        def _():
            out_ref[sl, :] = contrib

        @pl.when(e > 0)
        def _():
            out_ref[sl, :] = out_ref[sl, :] + contrib

    @pl.when(e >= n_exp)
    def _():
        g = _dot_nt(xb, swg_ref[...].astype(jnp.bfloat16))
        u = _dot_nt(xb, swu_ref[...].astype(jnp.bfloat16))
        h = (g * _silu(u)).astype(jnp.bfloat16)
        y = _dot_nt(h, swd_ref[...].astype(jnp.bfloat16))
        gate = _sigmoid(_dot_nt(xs, sg_ref[...]))
        out_ref[sl, :] = out_ref[sl, :] + y * gate


def kernel(hidden_states, gate_w, Wg, Wu, Wd, sWg, sWu, sWd, s_gate):
    x = hidden_states
    T, D = x.shape
    E, F, _ = Wg.shape
    S = sWg.shape[0]
    n_sh = S // F
    TBS = min(1024, T)
    ntb = T // TBS
    body = functools.partial(_fused_body, n_exp=E, tbs=TBS)
    return pl.pallas_call(
        body,
        grid=(E + n_sh, ntb),
        in_specs=[
            pl.BlockSpec((T, D), lambda e, tb: (0, 0)),
            pl.BlockSpec((E, D), lambda e, tb: (0, 0)),
            pl.BlockSpec((1, F, D), lambda e, tb: (jnp.minimum(e, E - 1), 0, 0)),
            pl.BlockSpec((1, F, D), lambda e, tb: (jnp.minimum(e, E - 1), 0, 0)),
            pl.BlockSpec((1, D, F), lambda e, tb: (jnp.minimum(e, E - 1), 0, 0)),
            pl.BlockSpec((F, D), lambda e, tb: (jnp.maximum(e - E, 0), 0)),
            pl.BlockSpec((F, D), lambda e, tb: (jnp.maximum(e - E, 0), 0)),
            pl.BlockSpec((D, F), lambda e, tb: (0, jnp.maximum(e - E, 0))),
            pl.BlockSpec((1, D), lambda e, tb: (0, 0)),
        ],
        out_specs=pl.BlockSpec((T, D), lambda e, tb: (0, 0)),
        out_shape=jax.ShapeDtypeStruct((T, D), jnp.float32),
        scratch_shapes=[pltpu.VMEM((T, E), jnp.float32)],
    )(x, gate_w, Wg, Wu, Wd, sWg, sWu, sWd, s_gate)
